# Initial kernel scaffold; baseline (speedup 1.0000x reference)
import jax
import jax.numpy as jnp
from jax.experimental import pallas as pl


def kernel(z, W):
    raise NotImplementedError("write your pallas kernel here")


# speedup vs baseline: 1.6168x; 1.6168x over previous
import functools

import jax
import jax.numpy as jnp
from jax.experimental import pallas as pl
from jax.experimental.pallas import tpu as pltpu

B, D, T, K = 16, 256, 1024, 1024
COMMITMENT_COST = 0.25


def _vq_body(z_ref, w_ref, wt_ref, zsq_ref, wsq_ref, codes_ref, zq_ref, loss_ref):
    z = z_ref[0]
    w = w_ref[...]
    wt = wt_ref[...]
    zsq = zsq_ref[0]
    wsq = wsq_ref[...]
    m = jax.lax.dot_general(w, z, (((1,), (0,)), ((), ())),
                            preferred_element_type=jnp.float32)
    dist = (zsq - 2.0 * m) + wsq
    mn = jnp.min(dist, axis=0, keepdims=True)
    iota_k = jax.lax.broadcasted_iota(jnp.int32, (K, T), 0)
    codes = jnp.min(jnp.where(dist == mn, iota_k, K), axis=0).astype(jnp.int32)
    codes_ref[0, 0, :] = codes
    onehot = (jax.lax.broadcasted_iota(jnp.int32, (K, T), 0)
              == codes[None, :]).astype(jnp.float32)
    zq = jax.lax.dot_general(wt, onehot, (((1,), (0,)), ((), ())),
                             preferred_element_type=jnp.float32,
                             precision=jax.lax.Precision.HIGHEST)
    zq_ref[0] = zq
    diff = zq - z
    loss_ref[0, 0, :] = jnp.full((128,), jnp.sum(diff * diff), jnp.float32)


@functools.partial(jax.jit, static_argnames=())
def kernel(z, W):
    Wt = W.T
    z_flat = jnp.transpose(z, (0, 2, 1)).reshape(-1, D)
    zsq = jnp.sum(z_flat ** 2, axis=1).reshape(B, 1, T)
    wsq = jnp.sum(W ** 2, axis=1).reshape(K, 1)
    codes3, zq, loss_parts = pl.pallas_call(
        _vq_body,
        grid=(B,),
        in_specs=[
            pl.BlockSpec((1, D, T), lambda b: (b, 0, 0)),
            pl.BlockSpec((K, D), lambda b: (0, 0)),
            pl.BlockSpec((D, K), lambda b: (0, 0)),
            pl.BlockSpec((1, 1, T), lambda b: (b, 0, 0)),
            pl.BlockSpec((K, 1), lambda b: (0, 0)),
        ],
        out_specs=[
            pl.BlockSpec((1, 1, T), lambda b: (b, 0, 0)),
            pl.BlockSpec((1, D, T), lambda b: (b, 0, 0)),
            pl.BlockSpec((1, 1, 128), lambda b: (b, 0, 0)),
        ],
        out_shape=[
            jax.ShapeDtypeStruct((B, 1, T), jnp.int32),
            jax.ShapeDtypeStruct((B, D, T), jnp.float32),
            jax.ShapeDtypeStruct((B, 1, 128), jnp.float32),
        ],
        compiler_params=pltpu.CompilerParams(
            dimension_semantics=("arbitrary",),
        ),
    )(z, W, Wt, zsq, wsq)
    codes = codes3.reshape(B * T)
    sq_err_sum = jnp.sum(loss_parts[:, 0, 0])
    vq_loss = (1.0 + COMMITMENT_COST) * sq_err_sum / (B * D * T)
    return zq, vq_loss, codes
